# Initial kernel scaffold; baseline (speedup 1.0000x reference)
import jax
import jax.numpy as jnp
from jax.experimental import pallas as pl


def kernel(x, edge_index, batch, Wl, bl, Wo1, bo1, Wo2, bo2, Wf, bf):
    raise NotImplementedError("write your pallas kernel here")


# speedup vs baseline: 4.1201x; 4.1201x over previous
import jax
import jax.numpy as jnp
from jax import lax
from jax.experimental import pallas as pl
from jax.experimental.pallas import tpu as pltpu
from jax.experimental.pallas import tpu_sc as plsc

N, E, D, C, G = 10000, 320000, 128, 10, 64
BN = 1000
NB = N // BN
CH = 128
E_PAD = 327680
CPT = 80
N_PAD = 10240
ZR = 128
SCH = 2048
NSCAN = E_PAD // SCH
RPT = 320
CORE_ROWS = 16 * RPT
SP_ROWS = CORE_ROWS + 8
OB = 4096
EPS = 1e-5


def _sc_body(h1_hbm, src_hbm, dst_hbm, out_hbm, src_v, dst_v, rows_v, agg_sh,
             sem):
    c = lax.axis_index("c")
    s = lax.axis_index("s")
    tid = c * 16 + s

    zero16 = jnp.zeros((16,), jnp.float32)

    def _zset(i, carry):
        rows_v[i // 8, pl.ds((i % 8) * 16, 16)] = zero16
        return carry

    lax.fori_loop(0, ZR * 8, _zset, 0)
    for k in range(N_PAD // 16 // ZR):
        pltpu.sync_copy(rows_v, agg_sh.at[pl.ds(s * (N_PAD // 16) + k * ZR, ZR)])

    pltpu.sync_copy(src_hbm.at[pl.ds(tid * CPT, CPT)], src_v)
    pltpu.sync_copy(dst_hbm.at[pl.ds(tid * CPT, CPT)], dst_v)
    plsc.subcore_barrier()

    def _chunk(j, carry):
        pltpu.async_copy(h1_hbm.at[src_v.at[j]], rows_v, sem).wait()
        pltpu.sync_copy(rows_v, agg_sh.at[dst_v.at[j]], add=True)
        return carry

    lax.fori_loop(0, CPT, _chunk, 0)
    plsc.subcore_barrier()

    WR = 624
    pltpu.sync_copy(agg_sh.at[pl.ds(s * WR, WR)],
                    out_hbm.at[pl.ds(c * N + s * WR, WR)])

    @pl.when(s == 0)
    def _():
        pltpu.sync_copy(agg_sh.at[pl.ds(16 * WR, N - 16 * WR)],
                        out_hbm.at[pl.ds(c * N + 16 * WR, N - 16 * WR)])


def _sc_propagate(h1, src_p, dst_p):
    mesh = plsc.VectorSubcoreMesh(core_axis_name="c", subcore_axis_name="s")
    f = pl.kernel(
        _sc_body,
        out_type=jax.ShapeDtypeStruct((2 * N, D), jnp.float32),
        mesh=mesh,
        scratch_types=[
            pltpu.VMEM((CPT, CH), jnp.int32),
            pltpu.VMEM((CPT, CH), jnp.int32),
            pltpu.VMEM((CH, D), jnp.float32),
            pltpu.VMEM_SHARED((N_PAD, D), jnp.float32),
            pltpu.SemaphoreType.DMA,
        ],
    )
    return f(h1, src_p, dst_p)


def _lin0_body(x_ref, w_ref, b_ref, o_ref):
    o_ref[...] = (jnp.dot(x_ref[...], w_ref[...],
                          preferred_element_type=jnp.float32) + b_ref[...])


def _tc_lin0(x, W, b2):
    return pl.pallas_call(
        _lin0_body,
        grid=(NB,),
        in_specs=[
            pl.BlockSpec((BN, D), lambda n: (n, 0)),
            pl.BlockSpec((D, D), lambda n: (0, 0)),
            pl.BlockSpec((1, D), lambda n: (0, 0)),
        ],
        out_specs=pl.BlockSpec((BN, D), lambda n: (n, 0)),
        out_shape=jax.ShapeDtypeStruct((N, D), jnp.float32),
    )(x, W, b2)


def _out_body(a_ref, h1_ref, w1_ref, b1_ref, w2_ref, b2_ref, o_ref):
    t = jnp.maximum(a_ref[...] + h1_ref[...], 0.0)
    t = jnp.maximum(jnp.dot(t, w1_ref[...], preferred_element_type=jnp.float32)
                    + b1_ref[...], 0.0)
    o_ref[...] = (jnp.dot(t, w2_ref[...], preferred_element_type=jnp.float32)
                  + b2_ref[...])


def _tc_out(agg, h1, W1, b1, W2, b2):
    return pl.pallas_call(
        _out_body,
        grid=(NB,),
        in_specs=[
            pl.BlockSpec((BN, D), lambda n: (n, 0)),
            pl.BlockSpec((BN, D), lambda n: (n, 0)),
            pl.BlockSpec((D, D), lambda n: (0, 0)),
            pl.BlockSpec((1, D), lambda n: (0, 0)),
            pl.BlockSpec((D, D), lambda n: (0, 0)),
            pl.BlockSpec((1, D), lambda n: (0, 0)),
        ],
        out_specs=pl.BlockSpec((BN, D), lambda n: (n, 0)),
        out_shape=jax.ShapeDtypeStruct((N, D), jnp.float32),
    )(agg, h1, W1, b1, W2, b2)


def _stats_body(o_ref, st_ref):
    mean = jnp.sum(o_ref[...], axis=0, keepdims=True) * (1.0 / N)
    dd = o_ref[...] - mean
    var = jnp.sum(dd * dd, axis=0, keepdims=True) / jnp.float32(N)
    st_ref[0:1, :] = mean
    st_ref[1:2, :] = var


def _tc_stats(out):
    return pl.pallas_call(
        _stats_body,
        in_specs=[pl.BlockSpec((N, D), lambda: (0, 0))],
        out_specs=pl.BlockSpec((8, D), lambda: (0, 0)),
        out_shape=jax.ShapeDtypeStruct((8, D), jnp.float32),
    )(out)


def _bnlin_body(o_ref, st_ref, w_ref, b_ref, h_ref):
    h = (o_ref[...] - st_ref[0:1, :]) / jnp.sqrt(st_ref[1:2, :] + EPS)
    h = jnp.maximum(h, 0.0)
    h_ref[...] = (jnp.dot(h, w_ref[...], preferred_element_type=jnp.float32)
                  + b_ref[...])


def _tc_bnlin(out, stats, W, b2):
    return pl.pallas_call(
        _bnlin_body,
        grid=(NB,),
        in_specs=[
            pl.BlockSpec((BN, D), lambda n: (n, 0)),
            pl.BlockSpec((8, D), lambda n: (0, 0)),
            pl.BlockSpec((D, D), lambda n: (0, 0)),
            pl.BlockSpec((1, D), lambda n: (0, 0)),
        ],
        out_specs=pl.BlockSpec((BN, D), lambda n: (n, 0)),
        out_shape=jax.ShapeDtypeStruct((N, D), jnp.float32),
    )(out, stats, W, b2)


def _pool_body(o_ref, st_ref, bt_ref, wf_ref, bf_ref, lg_ref, sums, cnts):
    n = pl.program_id(0)
    h = (o_ref[...] - st_ref[0:1, :]) / jnp.sqrt(st_ref[1:2, :] + EPS)
    bb = bt_ref[0, 0, :]
    gids = lax.broadcasted_iota(jnp.int32, (G, BN), 0)
    onehot = (bb[None, :] == gids).astype(jnp.float32)

    @pl.when(n == 0)
    def _():
        sums[...] = jnp.zeros_like(sums)
        cnts[...] = jnp.zeros_like(cnts)

    sums[...] += jnp.dot(onehot, h, preferred_element_type=jnp.float32,
                         precision=jax.lax.Precision.HIGHEST)
    cnts[...] += jnp.dot(onehot, jnp.ones((BN, D), jnp.float32),
                         preferred_element_type=jnp.float32,
                         precision=jax.lax.Precision.HIGHEST)

    @pl.when(n == NB - 1)
    def _():
        pooled = sums[...] / jnp.maximum(cnts[...], 1.0)
        lg_ref[...] = (jnp.dot(pooled, wf_ref[...],
                               preferred_element_type=jnp.float32) + bf_ref[...])


def _tc_pool(out, stats, batch3, Wf, bf2):
    return pl.pallas_call(
        _pool_body,
        grid=(NB,),
        in_specs=[
            pl.BlockSpec((BN, D), lambda n: (n, 0)),
            pl.BlockSpec((8, D), lambda n: (0, 0)),
            pl.BlockSpec((1, 1, BN), lambda n: (n, 0, 0)),
            pl.BlockSpec((D, C), lambda n: (0, 0)),
            pl.BlockSpec((1, C), lambda n: (0, 0)),
        ],
        out_specs=pl.BlockSpec((G, C), lambda n: (0, 0)),
        out_shape=jax.ShapeDtypeStruct((G, C), jnp.float32),
        scratch_shapes=[
            pltpu.VMEM((G, D), jnp.float32),
            pltpu.VMEM((G, D), jnp.float32),
        ],
    )(out, stats, batch3, Wf, bf2)


def _add2_body(a0_ref, a1_ref, o_ref):
    o_ref[...] = a0_ref[...] + a1_ref[...]


def _tc_add2(agg2):
    return pl.pallas_call(
        _add2_body,
        grid=(NB,),
        in_specs=[
            pl.BlockSpec((BN, D), lambda n: (n, 0)),
            pl.BlockSpec((BN, D), lambda n: (n + NB, 0)),
        ],
        out_specs=pl.BlockSpec((BN, D), lambda n: (n, 0)),
        out_shape=jax.ShapeDtypeStruct((N, D), jnp.float32),
    )(agg2, agg2)


def kernel(x, edge_index, batch, Wl, bl, Wo1, bo1, Wo2, bo2, Wf, bf):
    pad = E_PAD - E
    src_p = jnp.concatenate(
        [edge_index[0], jnp.zeros((pad,), jnp.int32)]).reshape(E_PAD // CH, CH)
    dst_p = jnp.concatenate(
        [edge_index[1], jnp.full((pad,), N, jnp.int32)]).reshape(E_PAD // CH, CH)
    batch3 = batch.reshape(NB, 1, BN)
    bl2 = bl.reshape(5, 1, D)
    bo1_2 = bo1.reshape(5, 1, D)
    bo2_2 = bo2.reshape(5, 1, D)
    bf2 = bf.reshape(1, C)

    h1 = _tc_lin0(x, Wl[0], bl2[0])
    out = stats = None
    for i in range(5):
        agg2 = _sc_propagate(h1, src_p, dst_p)
        agg = _tc_add2(agg2)
        out = _tc_out(agg, h1, Wo1[i], bo1_2[i], Wo2[i], bo2_2[i])
        stats = _tc_stats(out)
        if i < 4:
            h1 = _tc_bnlin(out, stats, Wl[i + 1], bl2[i + 1])
    return _tc_pool(out, stats, batch3, Wf, bf2)
